# baseline (speedup 1.0000x reference)
import functools

import jax
import jax.numpy as jnp
from jax import lax
from jax.experimental import pallas as pl
from jax.experimental.pallas import tpu as pltpu
from jax.experimental.pallas import tpu_sc as plsc

N = 10000
E = 160000
D = 128
H = 16
HC = 32

NC = 2
NS = 16
NW = NC * NS
NPAD = 10240
ROWS_PER_TILE = NPAD // NS
CH = 128
NCHUNK = E // CH
CHUNKS_PER_W = NCHUNK // NW
LEFTOVER = NCHUNK - CHUNKS_PER_W * NW
RB = 1000
NB = N // RB


def _tc_feats_body(x_ref, w_ref, a_ref,
                   h0_ref, h1_ref, h2_ref, h3_ref, att_ref, g_ref):
    xb = x_ref[...]
    hb = jnp.dot(xb, w_ref[...], preferred_element_type=jnp.float32)
    h0_ref[...] = hb[:, 0:128]
    h1_ref[...] = hb[:, 128:256]
    h2_ref[...] = hb[:, 256:384]
    h3_ref[...] = hb[:, 384:512]
    attb = jnp.dot(hb, a_ref[...], preferred_element_type=jnp.float32)
    att_ref[...] = attb
    gb = jnp.broadcast_to(jnp.max(attb, axis=0, keepdims=True), (8, 128))
    i = pl.program_id(0)

    @pl.when(i == 0)
    def _():
        g_ref[...] = gb

    @pl.when(i > 0)
    def _():
        g_ref[...] = jnp.maximum(g_ref[...], gb)


def _tc_feats(x, w, a_exp):
    din = x.shape[1]
    f32 = jnp.float32
    return pl.pallas_call(
        _tc_feats_body,
        grid=(NB,),
        in_specs=[
            pl.BlockSpec((RB, din), lambda i: (i, 0)),
            pl.BlockSpec((din, H * HC), lambda i: (0, 0)),
            pl.BlockSpec((H * HC, 128), lambda i: (0, 0)),
        ],
        out_specs=[
            pl.BlockSpec((RB, 128), lambda i: (i, 0)),
            pl.BlockSpec((RB, 128), lambda i: (i, 0)),
            pl.BlockSpec((RB, 128), lambda i: (i, 0)),
            pl.BlockSpec((RB, 128), lambda i: (i, 0)),
            pl.BlockSpec((RB, 128), lambda i: (i, 0)),
            pl.BlockSpec((8, 128), lambda i: (0, 0)),
        ],
        out_shape=[
            jax.ShapeDtypeStruct((N, 128), f32),
            jax.ShapeDtypeStruct((N, 128), f32),
            jax.ShapeDtypeStruct((N, 128), f32),
            jax.ShapeDtypeStruct((N, 128), f32),
            jax.ShapeDtypeStruct((N, 128), f32),
            jax.ShapeDtypeStruct((8, 128), f32),
        ],
    )(x, w, a_exp)


_MESH = plsc.VectorSubcoreMesh(core_axis_name="c", subcore_axis_name="s")


def _sc_logits_body(att_h, src_h, dst_h, g_h,
                    ex_h,
                    sidx, didx, ars, ard, exb, gv_ref, sem):
    cid = lax.axis_index("c")
    sid = lax.axis_index("s")
    wid = sid * NC + cid

    pltpu.sync_copy(g_h, gv_ref)
    gv = gv_ref[0, 0:16] + gv_ref[0, 16:32]

    def do_chunk(cix):
        base = cix * CH
        pltpu.sync_copy(src_h.at[pl.ds(base, CH)], sidx)
        pltpu.sync_copy(dst_h.at[pl.ds(base, CH)], didx)
        pltpu.async_copy(att_h.at[sidx], ars, sem).wait()
        pltpu.async_copy(att_h.at[didx], ard, sem).wait()

        @plsc.parallel_loop(0, CH, unroll=4)
        def edge(i):
            e = ars[i, 0:16] + ard[i, 16:32]
            e = jnp.where(e >= 0.0, e, 0.2 * e)
            exb[i, :] = jnp.exp(e - gv)
        pltpu.sync_copy(exb, ex_h.at[pl.ds(base, CH)])

    def chunk_loop(k, _):
        do_chunk(wid * CHUNKS_PER_W + k)
        return 0
    lax.fori_loop(0, CHUNKS_PER_W, chunk_loop, 0)

    @pl.when(wid < LEFTOVER)
    def _():
        do_chunk(NW * CHUNKS_PER_W + wid)


def _sc_edge_logits(att, src, dst, g):
    f32 = jnp.float32
    fn = pl.kernel(
        _sc_logits_body,
        out_type=jax.ShapeDtypeStruct((E, H), f32),
        mesh=_MESH,
        scratch_types=[
            pltpu.VMEM((CH,), jnp.int32),
            pltpu.VMEM((CH,), jnp.int32),
            pltpu.VMEM((CH, 128), f32),
            pltpu.VMEM((CH, 128), f32),
            pltpu.VMEM((CH, H), f32),
            pltpu.VMEM((8, 128), f32),
            pltpu.SemaphoreType.DMA,
        ],
    )
    return fn(att, src, dst, g)


NROUND = NPAD // 2
RPT2 = NROUND // NS


def _sc_messages_body(h0_h, h1_h, h2_h, h3_h, src_h, dst_h, ex_h,
                      outp_h,
                      sidx, didx, didx2, exb, hrows, msg, zb, outb, out_acc,
                      sem):
    cid = lax.axis_index("c")
    sid = lax.axis_index("s")
    wid = sid * NC + cid

    def zrow(i, _):
        for j in range(8):
            zb[i, 16 * j:16 * (j + 1)] = jnp.zeros((16,), jnp.float32)
        return 0
    lax.fori_loop(0, 64, zrow, 0)

    for c, hc_h in enumerate((h0_h, h1_h, h2_h, h3_h, None)):

        def do_chunk(cix, rbase):
            base = cix * CH
            pltpu.sync_copy(src_h.at[pl.ds(base, CH)], sidx)
            pltpu.sync_copy(dst_h.at[pl.ds(base, CH)], didx)
            pltpu.sync_copy(ex_h.at[pl.ds(base, CH)], exb)
            if hc_h is not None:
                pltpu.async_copy(hc_h.at[sidx], hrows, sem).wait()
            for v in range(8):
                d = didx[16 * v:16 * (v + 1)] - rbase
                m = (d >= 0) & (d < NROUND)
                didx2[16 * v:16 * (v + 1)] = jnp.where(m, d, NROUND)

            @plsc.parallel_loop(0, CH, unroll=2)
            def edge(i):
                exrow = exb[i, :]
                for j in range(8):
                    if hc_h is None:
                        msg[i, 16 * j:16 * (j + 1)] = exrow
                    else:
                        a = exrow[4 * c + j // 2]
                        msg[i, 16 * j:16 * (j + 1)] = (
                            hrows[i, 16 * j:16 * (j + 1)] * a)
            pltpu.sync_copy(msg, out_acc.at[didx2], add=True)

        def round_body(r, _):
            rbase = r * NROUND

            def zcp(k, _2):
                pltpu.sync_copy(zb, out_acc.at[pl.ds(sid * RPT2 + k * 64, 64)])
                return 0
            lax.fori_loop(0, RPT2 // 64, zcp, 0)
            plsc.subcore_barrier()

            def chunk_loop(k, _2):
                do_chunk(wid * CHUNKS_PER_W + k, rbase)
                return 0
            lax.fori_loop(0, CHUNKS_PER_W, chunk_loop, 0)

            @pl.when(wid < LEFTOVER)
            def _():
                do_chunk(NW * CHUNKS_PER_W + wid, rbase)

            plsc.subcore_barrier()

            def flush(k, _2):
                pltpu.sync_copy(out_acc.at[pl.ds(sid * RPT2 + k * 64, 64)],
                                outb)
                pltpu.sync_copy(
                    outb,
                    outp_h.at[pl.ds((cid * 5 + c) * NPAD + rbase
                                    + sid * RPT2 + k * 64, 64)])
                return 0
            lax.fori_loop(0, RPT2 // 64, flush, 0)
            plsc.subcore_barrier()
            return 0
        lax.fori_loop(0, 2, round_body, 0)


def _sc_messages(h0, h1, h2, h3, src, dst, ex):
    f32 = jnp.float32
    fn = pl.kernel(
        _sc_messages_body,
        out_type=jax.ShapeDtypeStruct((NC * 5 * NPAD, 128), f32),
        mesh=_MESH,
        scratch_types=[
            pltpu.VMEM((CH,), jnp.int32),
            pltpu.VMEM((CH,), jnp.int32),
            pltpu.VMEM((CH,), jnp.int32),
            pltpu.VMEM((CH, H), f32),
            pltpu.VMEM((CH, 128), f32),
            pltpu.VMEM((CH, 128), f32),
            pltpu.VMEM((64, 128), f32),
            pltpu.VMEM((64, 128), f32),
            pltpu.VMEM_SHARED((NROUND + 8, 128), f32),
            pltpu.SemaphoreType.DMA,
        ],
    )
    return fn(h0, h1, h2, h3, src, dst, ex)


def _tc_combine_body(p0_ref, p1_ref, b_ref, x_ref, *, last,
                     wp_ref=None, bp_ref=None):
    den = p0_ref[4][:, 0:16] + p1_ref[4][:, 0:16] + 1e-16
    acc = jnp.zeros((RB, HC), jnp.float32)
    for c in range(4):
        pc = p0_ref[c] + p1_ref[c]
        den4 = den[:, 4 * c:4 * (c + 1)]
        denr = jnp.broadcast_to(den4[:, :, None], (RB, 4, HC)).reshape(RB, 4 * HC)
        oc = jnp.maximum(pc / denr + b_ref[c][None, :], 0.0)
        acc = acc + oc.reshape(RB, 4, HC).sum(axis=1)
    xb = acc * (1.0 / H)
    if last:
        pen = jnp.dot(xb, wp_ref[...], preferred_element_type=jnp.float32)
        xb = xb * jnp.exp(pen + bp_ref[0, 0])
    x_ref[...] = xb


def _tc_combine(outp, b, wp=None, bp=None):
    f32 = jnp.float32
    p = outp.reshape(NC, 5, NPAD, 128)[:, :, :N]
    b4 = b.reshape(4, 128)
    last = wp is not None
    in_specs = [
        pl.BlockSpec((1, 5, RB, 128), lambda i: (0, 0, i, 0)),
        pl.BlockSpec((1, 5, RB, 128), lambda i: (1, 0, i, 0)),
        pl.BlockSpec((4, 128), lambda i: (0, 0)),
    ]
    args = [p, p, b4]
    if last:
        in_specs += [pl.BlockSpec((HC, 1), lambda i: (0, 0)),
                     pl.BlockSpec((1, 1), lambda i: (0, 0),
                                  memory_space=pltpu.SMEM)]
        args += [wp, bp.reshape(1, 1)]

    def body(*refs):
        if last:
            p0, p1, bb, wpr, bpr, xo = refs
            _tc_combine_body(p0.at[0], p1.at[0], bb, xo,
                             last=True, wp_ref=wpr, bp_ref=bpr)
        else:
            p0, p1, bb, xo = refs
            _tc_combine_body(p0.at[0], p1.at[0], bb, xo, last=False)

    return pl.pallas_call(
        body,
        grid=(NB,),
        in_specs=in_specs,
        out_specs=pl.BlockSpec((RB, HC), lambda i: (i, 0)),
        out_shape=jax.ShapeDtypeStruct((N, HC), f32),
    )(*args)


def _expand_attn(a_s, a_d):
    ch = a_s.shape[1]
    eye = jnp.eye(H, dtype=jnp.float32)
    blk_s = (a_s[:, :, None] * eye[:, None, :]).reshape(H * ch, H)
    blk_d = (a_d[:, :, None] * eye[:, None, :]).reshape(H * ch, H)
    pad = jnp.zeros((H * ch, 128 - 2 * H), jnp.float32)
    return jnp.concatenate([blk_s, blk_d, pad], axis=1)


def _gat_layer(x, w, a_s, a_d, src, dst):
    h0, h1, h2, h3, att, g = _tc_feats(x, w, _expand_attn(a_s, a_d))
    ex = _sc_edge_logits(att, src, dst, g)
    return _sc_messages(h0, h1, h2, h3, src, dst, ex)


def kernel(x, edge_index, W1, a1s, a1d, b1, W2, a2s, a2d, b2,
           W3, a3s, a3d, b3, Wp, bp):
    src = edge_index[0]
    dst = edge_index[1]
    outp = _gat_layer(x, W1, a1s, a1d, src, dst)
    x1 = _tc_combine(outp, b1)
    outp = _gat_layer(x1, W2, a2s, a2d, src, dst)
    x2 = _tc_combine(outp, b2)
    outp = _gat_layer(x2, W3, a3s, a3d, src, dst)
    return _tc_combine(outp, b3, Wp, bp)


# speedup vs baseline: 1.1955x; 1.1955x over previous
import functools

import jax
import jax.numpy as jnp
from jax import lax
from jax.experimental import pallas as pl
from jax.experimental.pallas import tpu as pltpu
from jax.experimental.pallas import tpu_sc as plsc

N = 10000
E = 160000
D = 128
H = 16
HC = 32

NC = 2
NS = 16
NW = NC * NS
NPAD = 10240
ROWS_PER_TILE = NPAD // NS
CH = 128
NCHUNK = E // CH
CHUNKS_PER_W = NCHUNK // NW
LEFTOVER = NCHUNK - CHUNKS_PER_W * NW
RB = 1000
NB = N // RB


def _tc_feats_body(x_ref, w_ref, a_ref,
                   h0_ref, h1_ref, h2_ref, h3_ref, att_ref, g_ref):
    xb = x_ref[...]
    hb = jnp.dot(xb, w_ref[...], preferred_element_type=jnp.float32)
    h0_ref[...] = hb[:, 0:128]
    h1_ref[...] = hb[:, 128:256]
    h2_ref[...] = hb[:, 256:384]
    h3_ref[...] = hb[:, 384:512]
    attb = jnp.dot(hb, a_ref[...], preferred_element_type=jnp.float32)
    att_ref[...] = attb
    gb = jnp.broadcast_to(jnp.max(attb, axis=0, keepdims=True), (8, 128))
    i = pl.program_id(0)

    @pl.when(i == 0)
    def _():
        g_ref[...] = gb

    @pl.when(i > 0)
    def _():
        g_ref[...] = jnp.maximum(g_ref[...], gb)


def _tc_feats(x, w, a_exp):
    din = x.shape[1]
    f32 = jnp.float32
    return pl.pallas_call(
        _tc_feats_body,
        grid=(NB,),
        in_specs=[
            pl.BlockSpec((RB, din), lambda i: (i, 0)),
            pl.BlockSpec((din, H * HC), lambda i: (0, 0)),
            pl.BlockSpec((H * HC, 128), lambda i: (0, 0)),
        ],
        out_specs=[
            pl.BlockSpec((RB, 128), lambda i: (i, 0)),
            pl.BlockSpec((RB, 128), lambda i: (i, 0)),
            pl.BlockSpec((RB, 128), lambda i: (i, 0)),
            pl.BlockSpec((RB, 128), lambda i: (i, 0)),
            pl.BlockSpec((RB, 128), lambda i: (i, 0)),
            pl.BlockSpec((8, 128), lambda i: (0, 0)),
        ],
        out_shape=[
            jax.ShapeDtypeStruct((N, 128), f32),
            jax.ShapeDtypeStruct((N, 128), f32),
            jax.ShapeDtypeStruct((N, 128), f32),
            jax.ShapeDtypeStruct((N, 128), f32),
            jax.ShapeDtypeStruct((N, 128), f32),
            jax.ShapeDtypeStruct((8, 128), f32),
        ],
    )(x, w, a_exp)


_MESH = plsc.VectorSubcoreMesh(core_axis_name="c", subcore_axis_name="s")

EPT = CHUNKS_PER_W * CH
GE = 2


def _sc_logits_body(att_h, src_h, dst_h, g_h,
                    ex_h,
                    sidxg, didxg, ars, ard, exb, gv_ref,
                    sem0, sem1, sem2, sem3):
    cid = lax.axis_index("c")
    sid = lax.axis_index("s")
    wid = sid * NC + cid

    pltpu.sync_copy(g_h, gv_ref)
    gv = gv_ref[0, 0:16] + gv_ref[0, 16:32]

    def do_group(gbase, nsub):
        ne = nsub * CH
        pltpu.sync_copy(src_h.at[pl.ds(gbase, ne)], sidxg.at[pl.ds(0, ne)])
        pltpu.sync_copy(dst_h.at[pl.ds(gbase, ne)], didxg.at[pl.ds(0, ne)])
        copies = []
        for t in range(nsub):
            copies.append(pltpu.async_copy(
                att_h.at[sidxg.at[pl.ds(t * CH, CH)]], ars.at[t],
                (sem0, sem1)[t]))
            copies.append(pltpu.async_copy(
                att_h.at[didxg.at[pl.ds(t * CH, CH)]], ard.at[t],
                (sem2, sem3)[t]))
        for t in range(nsub):
            copies[2 * t].wait()
            copies[2 * t + 1].wait()

            @plsc.parallel_loop(0, CH, unroll=4)
            def edge(i):
                e = ars[t, i, 0:16] + ard[t, i, 16:32]
                e = jnp.where(e >= 0.0, e, 0.2 * e)
                exb[i, :] = jnp.exp(e - gv)
            pltpu.sync_copy(exb, ex_h.at[pl.ds(gbase + t * CH, CH)])

    def group_loop(g, _):
        do_group(wid * EPT + g * (GE * CH), GE)
        return 0
    lax.fori_loop(0, CHUNKS_PER_W // GE, group_loop, 0)
    do_group(wid * EPT + (CHUNKS_PER_W // GE) * (GE * CH), CHUNKS_PER_W % GE)

    @pl.when(wid < LEFTOVER)
    def _():
        do_group(NW * EPT + wid * CH, 1)


def _sc_edge_logits(att, src, dst, g):
    f32 = jnp.float32
    fn = pl.kernel(
        _sc_logits_body,
        out_type=jax.ShapeDtypeStruct((E, H), f32),
        mesh=_MESH,
        scratch_types=[
            pltpu.VMEM((GE * CH,), jnp.int32),
            pltpu.VMEM((GE * CH,), jnp.int32),
            pltpu.VMEM((GE, CH, 128), f32),
            pltpu.VMEM((GE, CH, 128), f32),
            pltpu.VMEM((CH, H), f32),
            pltpu.VMEM((8, 128), f32),
            pltpu.SemaphoreType.DMA,
            pltpu.SemaphoreType.DMA,
            pltpu.SemaphoreType.DMA,
            pltpu.SemaphoreType.DMA,
        ],
    )
    return fn(att, src, dst, g)


NROUND = NPAD // 2
RPT2 = NROUND // NS


def _sc_messages_body(h0_h, h1_h, h2_h, h3_h, src_h, dst_h, ex_h,
                      outp_h,
                      sidxg, didxg, didx2, exb, hrows, msg, zb, outb, out_acc,
                      sem0, sem1, sem2, sem3):
    cid = lax.axis_index("c")
    sid = lax.axis_index("s")
    wid = sid * NC + cid
    sems = (sem0, sem1, sem2, sem3)

    def zrow(i, _):
        for j in range(8):
            zb[i, 16 * j:16 * (j + 1)] = jnp.zeros((16,), jnp.float32)
        return 0
    lax.fori_loop(0, 32, zrow, 0)

    for c, hc_h in enumerate((h0_h, h1_h, h2_h, h3_h, None)):

        def do_group(gbase, nsub, rbase):
            ne = nsub * CH
            pltpu.sync_copy(src_h.at[pl.ds(gbase, ne)], sidxg.at[pl.ds(0, ne)])
            pltpu.sync_copy(dst_h.at[pl.ds(gbase, ne)], didxg.at[pl.ds(0, ne)])
            if hc_h is not None:
                copies = []
                for t in range(nsub):
                    copies.append(pltpu.async_copy(
                        hc_h.at[sidxg.at[pl.ds(t * CH, CH)]],
                        hrows.at[t], sems[t]))
            for t in range(nsub):
                pltpu.sync_copy(ex_h.at[pl.ds(gbase + t * CH, CH)], exb)
                if hc_h is not None:
                    copies[t].wait()
                for v in range(8):
                    d = didxg[t * CH + 16 * v:t * CH + 16 * (v + 1)] - rbase
                    m = (d >= 0) & (d < NROUND)
                    didx2[16 * v:16 * (v + 1)] = jnp.where(m, d, NROUND)

                @plsc.parallel_loop(0, CH, unroll=2)
                def edge(i):
                    exrow = exb[i, :]
                    for j in range(8):
                        if hc_h is None:
                            msg[i, 16 * j:16 * (j + 1)] = exrow
                        else:
                            a = exrow[4 * c + j // 2]
                            msg[i, 16 * j:16 * (j + 1)] = (
                                hrows[t, i, 16 * j:16 * (j + 1)] * a)
                pltpu.sync_copy(msg, out_acc.at[didx2], add=True)

        def round_body(r, _):
            rbase = r * NROUND

            def zcp(k, _2):
                pltpu.sync_copy(zb, out_acc.at[pl.ds(sid * RPT2 + k * 32, 32)])
                return 0
            lax.fori_loop(0, RPT2 // 32, zcp, 0)
            plsc.subcore_barrier()

            def group_loop(g, _2):
                do_group(wid * EPT + g * (GE * CH), GE, rbase)
                return 0
            lax.fori_loop(0, CHUNKS_PER_W // GE, group_loop, 0)
            do_group(wid * EPT + (CHUNKS_PER_W // GE) * (GE * CH),
                     CHUNKS_PER_W % GE, rbase)

            @pl.when(wid < LEFTOVER)
            def _():
                do_group(NW * EPT + wid * CH, 1, rbase)

            plsc.subcore_barrier()

            def flush(k, _2):
                pltpu.sync_copy(out_acc.at[pl.ds(sid * RPT2 + k * 32, 32)],
                                outb)
                pltpu.sync_copy(
                    outb,
                    outp_h.at[pl.ds((cid * 5 + c) * NPAD + rbase
                                    + sid * RPT2 + k * 32, 32)])
                return 0
            lax.fori_loop(0, RPT2 // 32, flush, 0)
            plsc.subcore_barrier()
            return 0
        lax.fori_loop(0, 2, round_body, 0)


def _sc_messages(h0, h1, h2, h3, src, dst, ex):
    f32 = jnp.float32
    fn = pl.kernel(
        _sc_messages_body,
        out_type=jax.ShapeDtypeStruct((NC * 5 * NPAD, 128), f32),
        mesh=_MESH,
        scratch_types=[
            pltpu.VMEM((GE * CH,), jnp.int32),
            pltpu.VMEM((GE * CH,), jnp.int32),
            pltpu.VMEM((CH,), jnp.int32),
            pltpu.VMEM((CH, H), f32),
            pltpu.VMEM((GE, CH, 128), f32),
            pltpu.VMEM((CH, 128), f32),
            pltpu.VMEM((32, 128), f32),
            pltpu.VMEM((32, 128), f32),
            pltpu.VMEM_SHARED((NROUND + 8, 128), f32),
            pltpu.SemaphoreType.DMA,
            pltpu.SemaphoreType.DMA,
            pltpu.SemaphoreType.DMA,
            pltpu.SemaphoreType.DMA,
        ],
    )
    return fn(h0, h1, h2, h3, src, dst, ex)


def _tc_combine_body(p0_ref, p1_ref, b_ref, x_ref, *, last,
                     wp_ref=None, bp_ref=None):
    den = p0_ref[4][:, 0:16] + p1_ref[4][:, 0:16] + 1e-16
    acc = jnp.zeros((RB, HC), jnp.float32)
    for c in range(4):
        pc = p0_ref[c] + p1_ref[c]
        den4 = den[:, 4 * c:4 * (c + 1)]
        denr = jnp.broadcast_to(den4[:, :, None], (RB, 4, HC)).reshape(RB, 4 * HC)
        oc = jnp.maximum(pc / denr + b_ref[c][None, :], 0.0)
        acc = acc + oc.reshape(RB, 4, HC).sum(axis=1)
    xb = acc * (1.0 / H)
    if last:
        pen = jnp.dot(xb, wp_ref[...], preferred_element_type=jnp.float32)
        xb = xb * jnp.exp(pen + bp_ref[0, 0])
    x_ref[...] = xb


def _tc_combine(outp, b, wp=None, bp=None):
    f32 = jnp.float32
    p = outp.reshape(NC, 5, NPAD, 128)[:, :, :N]
    b4 = b.reshape(4, 128)
    last = wp is not None
    in_specs = [
        pl.BlockSpec((1, 5, RB, 128), lambda i: (0, 0, i, 0)),
        pl.BlockSpec((1, 5, RB, 128), lambda i: (1, 0, i, 0)),
        pl.BlockSpec((4, 128), lambda i: (0, 0)),
    ]
    args = [p, p, b4]
    if last:
        in_specs += [pl.BlockSpec((HC, 1), lambda i: (0, 0)),
                     pl.BlockSpec((1, 1), lambda i: (0, 0),
                                  memory_space=pltpu.SMEM)]
        args += [wp, bp.reshape(1, 1)]

    def body(*refs):
        if last:
            p0, p1, bb, wpr, bpr, xo = refs
            _tc_combine_body(p0.at[0], p1.at[0], bb, xo,
                             last=True, wp_ref=wpr, bp_ref=bpr)
        else:
            p0, p1, bb, xo = refs
            _tc_combine_body(p0.at[0], p1.at[0], bb, xo, last=False)

    return pl.pallas_call(
        body,
        grid=(NB,),
        in_specs=in_specs,
        out_specs=pl.BlockSpec((RB, HC), lambda i: (i, 0)),
        out_shape=jax.ShapeDtypeStruct((N, HC), f32),
    )(*args)


def _expand_attn(a_s, a_d):
    ch = a_s.shape[1]
    eye = jnp.eye(H, dtype=jnp.float32)
    blk_s = (a_s[:, :, None] * eye[:, None, :]).reshape(H * ch, H)
    blk_d = (a_d[:, :, None] * eye[:, None, :]).reshape(H * ch, H)
    pad = jnp.zeros((H * ch, 128 - 2 * H), jnp.float32)
    return jnp.concatenate([blk_s, blk_d, pad], axis=1)


def _gat_layer(x, w, a_s, a_d, src, dst):
    h0, h1, h2, h3, att, g = _tc_feats(x, w, _expand_attn(a_s, a_d))
    ex = _sc_edge_logits(att, src, dst, g)
    return _sc_messages(h0, h1, h2, h3, src, dst, ex)


def kernel(x, edge_index, W1, a1s, a1d, b1, W2, a2s, a2d, b2,
           W3, a3s, a3d, b3, Wp, bp):
    src = edge_index[0]
    dst = edge_index[1]
    outp = _gat_layer(x, W1, a1s, a1d, src, dst)
    x1 = _tc_combine(outp, b1)
    outp = _gat_layer(x1, W2, a2s, a2d, src, dst)
    x2 = _tc_combine(outp, b2)
    outp = _gat_layer(x2, W3, a3s, a3d, src, dst)
    return _tc_combine(outp, b3, Wp, bp)
